# Initial kernel scaffold; baseline (speedup 1.0000x reference)
import jax
import jax.numpy as jnp
from jax.experimental import pallas as pl


def kernel(x_p, edge_index_p, x_p_batch, W_conv0, b_conv0, W_conv1, b_conv1, W_conv2, b_conv2, W_lin0, b_lin0, W_lin1, b_lin1):
    raise NotImplementedError("write your pallas kernel here")


# speedup vs baseline: 18.0054x; 18.0054x over previous
import functools

import jax
import jax.numpy as jnp
from jax import lax
from jax.experimental import pallas as pl
from jax.experimental.pallas import tpu as pltpu
from jax.experimental.pallas import tpu_sc as plsc

N = 10000
E = 320000
D = 128
H = 128
HH = H // 2
G = 64

N_PAD = 10240
CHUNK = 125
IDX_ROWS = E // CHUNK
DEG_ROWS = IDX_ROWS // 32
AGG_ROWS = IDX_ROWS // 16
TILE_SLICE = N_PAD // 16
BLK = 400
NBLK = N // BLK

_mesh = plsc.VectorSubcoreMesh(core_axis_name="c", subcore_axis_name="s")
_sc_params = pltpu.CompilerParams(use_tc_tiling_on_sc=False)


def _sc_degree(dst2d, ones_c, zeros16):
    @functools.partial(
        pl.kernel,
        mesh=_mesh,
        out_type=jax.ShapeDtypeStruct((2, N_PAD, 16), jnp.float32),
        compiler_params=_sc_params,
        scratch_types=[
            pltpu.VMEM((DEG_ROWS, CHUNK), jnp.int32),
            pltpu.VMEM((CHUNK, 16), jnp.float32),
            pltpu.VMEM_SHARED((N_PAD, 16), jnp.float32),
        ],
    )
    def k(dst_hbm, ones_hbm, z_hbm, out_hbm, didx, ones_v, acc):
        cid = lax.axis_index("c")
        sid = lax.axis_index("s")
        wrow = (cid * 16 + sid) * DEG_ROWS
        pltpu.sync_copy(dst_hbm.at[pl.ds(wrow, DEG_ROWS)], didx)
        pltpu.sync_copy(ones_hbm, ones_v)
        pltpu.sync_copy(z_hbm, acc.at[pl.ds(sid * TILE_SLICE, TILE_SLICE)])
        plsc.subcore_barrier()

        @pl.loop(0, DEG_ROWS)
        def _(r):
            pltpu.sync_copy(ones_v, acc.at[didx.at[r]], add=True)

        plsc.subcore_barrier()
        pltpu.sync_copy(
            acc.at[pl.ds(sid * TILE_SLICE, TILE_SLICE)],
            out_hbm.at[cid, pl.ds(sid * TILE_SLICE, TILE_SLICE)],
        )

    return k(dst2d, ones_c, zeros16)


def _sc_aggregate(y2, src2d, dst2d, zeros64):
    @functools.partial(
        pl.kernel,
        mesh=_mesh,
        out_type=jax.ShapeDtypeStruct((2, N_PAD, HH), jnp.float32),
        compiler_params=_sc_params,
        scratch_types=[
            pltpu.VMEM((AGG_ROWS, CHUNK), jnp.int32),
            pltpu.VMEM((AGG_ROWS, CHUNK), jnp.int32),
            pltpu.VMEM((CHUNK, HH), jnp.float32),
            pltpu.VMEM((CHUNK, HH), jnp.float32),
            pltpu.VMEM_SHARED((N_PAD, HH), jnp.float32),
            pltpu.SemaphoreType.DMA,
            pltpu.SemaphoreType.DMA,
        ],
    )
    def k(y_hbm, src_hbm, dst_hbm, z_hbm, out_hbm,
          sidx, didx, bufa, bufb, acc, sema, semb):
        cid = lax.axis_index("c")
        sid = lax.axis_index("s")
        wrow = sid * AGG_ROWS
        yc = y_hbm.at[cid]
        pltpu.sync_copy(src_hbm.at[pl.ds(wrow, AGG_ROWS)], sidx)
        pltpu.sync_copy(dst_hbm.at[pl.ds(wrow, AGG_ROWS)], didx)
        pltpu.sync_copy(z_hbm, acc.at[pl.ds(sid * TILE_SLICE, TILE_SLICE)])
        plsc.subcore_barrier()

        pltpu.async_copy(yc.at[sidx.at[0]], bufa, sema)

        @pl.loop(0, AGG_ROWS // 2)
        def _(j):
            r = j * 2
            pltpu.async_copy(yc.at[sidx.at[r + 1]], bufb, semb)
            pltpu.make_async_copy(yc.at[sidx.at[r]], bufa, sema).wait()
            pltpu.sync_copy(bufa, acc.at[didx.at[r]], add=True)

            @pl.when(r + 2 < AGG_ROWS)
            def _():
                pltpu.async_copy(yc.at[sidx.at[r + 2]], bufa, sema)

            pltpu.make_async_copy(yc.at[sidx.at[r + 1]], bufb, semb).wait()
            pltpu.sync_copy(bufb, acc.at[didx.at[r + 1]], add=True)

        plsc.subcore_barrier()
        pltpu.sync_copy(
            acc.at[pl.ds(sid * TILE_SLICE, TILE_SLICE)],
            out_hbm.at[cid, pl.ds(sid * TILE_SLICE, TILE_SLICE)],
        )

    return k(y2, src2d, dst2d, zeros64)


def _dot(a, b):
    return lax.dot_general(a, b, (((1,), (0,)), ((), ())),
                           precision=lax.Precision.HIGHEST,
                           preferred_element_type=jnp.float32)


def _split(o_ref, y):
    o_ref[0] = y[:, :HH]
    o_ref[1] = y[:, HH:]


def _tc_matmul(x, w):
    def body(x_ref, w_ref, o_ref):
        o_ref[...] = _dot(x_ref[...], w_ref[...])

    return pl.pallas_call(
        body,
        grid=(NBLK,),
        in_specs=[
            pl.BlockSpec((BLK, D), lambda i: (i, 0)),
            pl.BlockSpec((D, H), lambda i: (0, 0)),
        ],
        out_specs=pl.BlockSpec((BLK, H), lambda i: (i, 0)),
        out_shape=jax.ShapeDtypeStruct((N, H), jnp.float32),
    )(x, w)


def _tc_scale(xw, degp):
    def body(xw_ref, d_ref, o_ref):
        deg = d_ref[0, :, :1] + d_ref[1, :, :1] + 1.0
        _split(o_ref, lax.rsqrt(deg) * xw_ref[...])

    return pl.pallas_call(
        body,
        grid=(NBLK,),
        in_specs=[
            pl.BlockSpec((BLK, H), lambda i: (i, 0)),
            pl.BlockSpec((2, BLK, 16), lambda i: (0, i, 0)),
        ],
        out_specs=pl.BlockSpec((2, BLK, HH), lambda i: (0, i, 0)),
        out_shape=jax.ShapeDtypeStruct((2, N, HH), jnp.float32),
    )(xw, degp)


def _tc_layer(parts, y2, degp, b, w_next):
    def body(p_ref, y_ref, d_ref, b_ref, w_ref, o_ref):
        deg = d_ref[0, :, :1] + d_ref[1, :, :1] + 1.0
        dinv = lax.rsqrt(deg)
        s = jnp.concatenate([p_ref[0] + y_ref[0], p_ref[1] + y_ref[1]], axis=1)
        h = jnp.maximum(dinv * s + b_ref[...], 0.0)
        _split(o_ref, dinv * _dot(h, w_ref[...]))

    return pl.pallas_call(
        body,
        grid=(NBLK,),
        in_specs=[
            pl.BlockSpec((2, BLK, HH), lambda i: (0, i, 0)),
            pl.BlockSpec((2, BLK, HH), lambda i: (0, i, 0)),
            pl.BlockSpec((2, BLK, 16), lambda i: (0, i, 0)),
            pl.BlockSpec((1, H), lambda i: (0, 0)),
            pl.BlockSpec((H, H), lambda i: (0, 0)),
        ],
        out_specs=pl.BlockSpec((2, BLK, HH), lambda i: (0, i, 0)),
        out_shape=jax.ShapeDtypeStruct((2, N, HH), jnp.float32),
    )(parts, y2, degp, b, w_next)


def _tc_final(parts, y2, degp, b, batch2d, w0, b0, w1, b1):
    def body(p_ref, y_ref, d_ref, b_ref, bat_ref, w0_ref, b0_ref, w1_ref,
             b1_ref, o_ref, acc_ref):
        i = pl.program_id(0)

        @pl.when(i == 0)
        def _():
            acc_ref[...] = jnp.full((G, H), -jnp.inf, jnp.float32)

        deg = d_ref[0, :, :1] + d_ref[1, :, :1] + 1.0
        dinv = lax.rsqrt(deg)
        s = jnp.concatenate([p_ref[0] + y_ref[0], p_ref[1] + y_ref[1]], axis=1)
        h = jnp.maximum(dinv * s + b_ref[...], 0.0)
        bat = bat_ref[...]
        m = acc_ref[...]
        rows = []
        for g in range(G):
            sel = jnp.where(bat == g, h, -jnp.inf)
            rows.append(jnp.max(sel, axis=0, keepdims=True))
        m = jnp.maximum(m, jnp.concatenate(rows, axis=0))
        acc_ref[...] = m

        @pl.when(i == NBLK - 1)
        def _():
            g1 = _dot(m, w0_ref[...]) + b0_ref[...]
            o_ref[...] = _dot(g1, w1_ref[...]) + b1_ref[...]

    return pl.pallas_call(
        body,
        grid=(NBLK,),
        in_specs=[
            pl.BlockSpec((2, BLK, HH), lambda i: (0, i, 0)),
            pl.BlockSpec((2, BLK, HH), lambda i: (0, i, 0)),
            pl.BlockSpec((2, BLK, 16), lambda i: (0, i, 0)),
            pl.BlockSpec((1, H), lambda i: (0, 0)),
            pl.BlockSpec((BLK, 1), lambda i: (i, 0)),
            pl.BlockSpec((H, H), lambda i: (0, 0)),
            pl.BlockSpec((1, H), lambda i: (0, 0)),
            pl.BlockSpec((H, 1), lambda i: (0, 0)),
            pl.BlockSpec((1, 1), lambda i: (0, 0)),
        ],
        out_specs=pl.BlockSpec((G, 1), lambda i: (0, 0)),
        out_shape=jax.ShapeDtypeStruct((G, 1), jnp.float32),
        scratch_shapes=[pltpu.VMEM((G, H), jnp.float32)],
    )(parts, y2, degp, b, batch2d, w0, b0, w1, b1)


def kernel(x_p, edge_index_p, x_p_batch, W_conv0, b_conv0, W_conv1, b_conv1,
           W_conv2, b_conv2, W_lin0, b_lin0, W_lin1, b_lin1):
    src2d = edge_index_p[0].reshape(IDX_ROWS, CHUNK)
    dst2d = edge_index_p[1].reshape(IDX_ROWS, CHUNK)
    batch2d = x_p_batch.reshape(N, 1)
    ones_c = jnp.ones((CHUNK, 16), jnp.float32)
    zeros16 = jnp.zeros((TILE_SLICE, 16), jnp.float32)
    zeros64 = jnp.zeros((TILE_SLICE, HH), jnp.float32)
    b0 = b_conv0.reshape(1, H)
    b1 = b_conv1.reshape(1, H)
    b2 = b_conv2.reshape(1, H)
    bl0 = b_lin0.reshape(1, H)
    bl1 = b_lin1.reshape(1, 1)

    degp = _sc_degree(dst2d, ones_c, zeros16)
    xw0 = _tc_matmul(x_p, W_conv0)
    y0 = _tc_scale(xw0, degp)
    p0 = _sc_aggregate(y0, src2d, dst2d, zeros64)
    y1 = _tc_layer(p0, y0, degp, b0, W_conv1)
    p1 = _sc_aggregate(y1, src2d, dst2d, zeros64)
    y2 = _tc_layer(p1, y1, degp, b1, W_conv2)
    p2 = _sc_aggregate(y2, src2d, dst2d, zeros64)
    return _tc_final(p2, y2, degp, b2, batch2d, W_lin0, bl0, W_lin1, bl1)


# speedup vs baseline: 20.1095x; 1.1169x over previous
import functools

import jax
import jax.numpy as jnp
from jax import lax
from jax.experimental import pallas as pl
from jax.experimental.pallas import tpu as pltpu
from jax.experimental.pallas import tpu_sc as plsc

N = 10000
E = 320000
D = 128
H = 128
HH = H // 2
G = 64

N_PAD = 10240
CHUNK = 125
IDX_ROWS = E // CHUNK
DEG_ROWS = IDX_ROWS // 32
AGG_ROWS = IDX_ROWS // 16
TILE_SLICE = N_PAD // 16
BLK = 400
NBLK = N // BLK

_mesh = plsc.VectorSubcoreMesh(core_axis_name="c", subcore_axis_name="s")
_sc_params = pltpu.CompilerParams(use_tc_tiling_on_sc=False)


def _sc_degree(dst2d, ones_c, zeros16):
    @functools.partial(
        pl.kernel,
        mesh=_mesh,
        out_type=jax.ShapeDtypeStruct((2, N_PAD, 16), jnp.float32),
        compiler_params=_sc_params,
        scratch_types=[
            pltpu.VMEM((DEG_ROWS, CHUNK), jnp.int32),
            pltpu.VMEM((CHUNK, 16), jnp.float32),
            pltpu.VMEM_SHARED((N_PAD, 16), jnp.float32),
        ],
    )
    def k(dst_hbm, ones_hbm, z_hbm, out_hbm, didx, ones_v, acc):
        cid = lax.axis_index("c")
        sid = lax.axis_index("s")
        wrow = (cid * 16 + sid) * DEG_ROWS
        pltpu.sync_copy(dst_hbm.at[pl.ds(wrow, DEG_ROWS)], didx)
        pltpu.sync_copy(ones_hbm, ones_v)
        pltpu.sync_copy(z_hbm, acc.at[pl.ds(sid * TILE_SLICE, TILE_SLICE)])
        plsc.subcore_barrier()

        @pl.loop(0, DEG_ROWS)
        def _(r):
            pltpu.sync_copy(ones_v, acc.at[didx.at[r]], add=True)

        plsc.subcore_barrier()
        pltpu.sync_copy(
            acc.at[pl.ds(sid * TILE_SLICE, TILE_SLICE)],
            out_hbm.at[cid, pl.ds(sid * TILE_SLICE, TILE_SLICE)],
        )

    return k(dst2d, ones_c, zeros16)


def _sc_aggregate(y2, src2d, dst2d, zeros64):
    @functools.partial(
        pl.kernel,
        mesh=_mesh,
        out_type=jax.ShapeDtypeStruct((2, N_PAD, HH), jnp.float32),
        compiler_params=_sc_params,
        scratch_types=[
            pltpu.VMEM((AGG_ROWS, CHUNK), jnp.int32),
            pltpu.VMEM((AGG_ROWS, CHUNK), jnp.int32),
            pltpu.VMEM((CHUNK, HH), jnp.float32),
            pltpu.VMEM((CHUNK, HH), jnp.float32),
            pltpu.VMEM_SHARED((N_PAD, HH), jnp.float32),
            pltpu.SemaphoreType.DMA,
            pltpu.SemaphoreType.DMA,
        ],
    )
    def k(y_hbm, src_hbm, dst_hbm, z_hbm, out_hbm,
          sidx, didx, bufa, bufb, acc, sema, semb):
        cid = lax.axis_index("c")
        sid = lax.axis_index("s")
        wrow = sid * AGG_ROWS
        yc = y_hbm.at[cid]
        pltpu.sync_copy(src_hbm.at[pl.ds(wrow, AGG_ROWS)], sidx)
        pltpu.sync_copy(dst_hbm.at[pl.ds(wrow, AGG_ROWS)], didx)
        pltpu.sync_copy(z_hbm, acc.at[pl.ds(sid * TILE_SLICE, TILE_SLICE)])
        plsc.subcore_barrier()

        pltpu.async_copy(yc.at[sidx.at[0]], bufa, sema)

        @pl.loop(0, AGG_ROWS // 2)
        def _(j):
            r = j * 2
            pltpu.async_copy(yc.at[sidx.at[r + 1]], bufb, semb)
            pltpu.make_async_copy(yc.at[sidx.at[r]], bufa, sema).wait()
            pltpu.sync_copy(bufa, acc.at[didx.at[r]], add=True)

            @pl.when(r + 2 < AGG_ROWS)
            def _():
                pltpu.async_copy(yc.at[sidx.at[r + 2]], bufa, sema)

            pltpu.make_async_copy(yc.at[sidx.at[r + 1]], bufb, semb).wait()
            pltpu.sync_copy(bufb, acc.at[didx.at[r + 1]], add=True)

        plsc.subcore_barrier()
        pltpu.sync_copy(
            acc.at[pl.ds(sid * TILE_SLICE, TILE_SLICE)],
            out_hbm.at[cid, pl.ds(sid * TILE_SLICE, TILE_SLICE)],
        )

    return k(y2, src2d, dst2d, zeros64)


def _bdot(a, b):
    return lax.dot_general(a.astype(jnp.bfloat16), b.astype(jnp.bfloat16),
                           (((1,), (0,)), ((), ())),
                           preferred_element_type=jnp.float32)


def _dot_f32(a, b):
    return lax.dot_general(a, b, (((1,), (0,)), ((), ())),
                           precision=lax.Precision.HIGHEST,
                           preferred_element_type=jnp.float32)


def _split(o_ref, y):
    o_ref[0] = y[:, :HH]
    o_ref[1] = y[:, HH:]


def _tc_matmul(x, w):
    def body(x_ref, w_ref, o_ref):
        o_ref[...] = _bdot(x_ref[...], w_ref[...])

    return pl.pallas_call(
        body,
        grid=(NBLK,),
        in_specs=[
            pl.BlockSpec((BLK, D), lambda i: (i, 0)),
            pl.BlockSpec((D, H), lambda i: (0, 0)),
        ],
        out_specs=pl.BlockSpec((BLK, H), lambda i: (i, 0)),
        out_shape=jax.ShapeDtypeStruct((N, H), jnp.float32),
    )(x, w)


def _tc_scale(xw, degp):
    def body(xw_ref, d_ref, o_ref):
        deg = d_ref[0, :, :1] + d_ref[1, :, :1] + 1.0
        _split(o_ref, lax.rsqrt(deg) * xw_ref[...])

    return pl.pallas_call(
        body,
        grid=(NBLK,),
        in_specs=[
            pl.BlockSpec((BLK, H), lambda i: (i, 0)),
            pl.BlockSpec((2, BLK, 16), lambda i: (0, i, 0)),
        ],
        out_specs=pl.BlockSpec((2, BLK, HH), lambda i: (0, i, 0)),
        out_shape=jax.ShapeDtypeStruct((2, N, HH), jnp.float32),
    )(xw, degp)


def _tc_layer(parts, y2, degp, b, w_next):
    def body(p_ref, y_ref, d_ref, b_ref, w_ref, o_ref):
        deg = d_ref[0, :, :1] + d_ref[1, :, :1] + 1.0
        dinv = lax.rsqrt(deg)
        s = jnp.concatenate([p_ref[0] + y_ref[0], p_ref[1] + y_ref[1]], axis=1)
        h = jnp.maximum(dinv * s + b_ref[...], 0.0)
        _split(o_ref, dinv * _bdot(h, w_ref[...]))

    return pl.pallas_call(
        body,
        grid=(NBLK,),
        in_specs=[
            pl.BlockSpec((2, BLK, HH), lambda i: (0, i, 0)),
            pl.BlockSpec((2, BLK, HH), lambda i: (0, i, 0)),
            pl.BlockSpec((2, BLK, 16), lambda i: (0, i, 0)),
            pl.BlockSpec((1, H), lambda i: (0, 0)),
            pl.BlockSpec((H, H), lambda i: (0, 0)),
        ],
        out_specs=pl.BlockSpec((2, BLK, HH), lambda i: (0, i, 0)),
        out_shape=jax.ShapeDtypeStruct((2, N, HH), jnp.float32),
    )(parts, y2, degp, b, w_next)


def _tc_final(parts, y2, degp, b, batch2d, w0, b0, w1, b1):
    def body(p_ref, y_ref, d_ref, b_ref, bat_ref, w0_ref, b0_ref, w1_ref,
             b1_ref, o_ref, acc_ref):
        i = pl.program_id(0)

        @pl.when(i == 0)
        def _():
            acc_ref[...] = jnp.full((G, H), -jnp.inf, jnp.float32)

        deg = d_ref[0, :, :1] + d_ref[1, :, :1] + 1.0
        dinv = lax.rsqrt(deg)
        s = jnp.concatenate([p_ref[0] + y_ref[0], p_ref[1] + y_ref[1]], axis=1)
        h = jnp.maximum(dinv * s + b_ref[...], 0.0)
        bat = bat_ref[...]
        g_lo = bat_ref[0, 0]
        g_hi = bat_ref[BLK - 1, 0]
        seg_col = lax.broadcasted_iota(jnp.int32, (G, 1), 0)

        def seg_body(g, m):
            sel = jnp.where(bat == g, h, -jnp.inf)
            row = jnp.max(sel, axis=0, keepdims=True)
            return jnp.maximum(m, jnp.where(seg_col == g, row, -jnp.inf))

        acc_ref[...] = lax.fori_loop(g_lo, g_hi + 1, seg_body, acc_ref[...])

        @pl.when(i == NBLK - 1)
        def _():
            g1 = _dot_f32(acc_ref[...], w0_ref[...]) + b0_ref[...]
            o_ref[...] = _dot_f32(g1, w1_ref[...]) + b1_ref[...]

    return pl.pallas_call(
        body,
        grid=(NBLK,),
        in_specs=[
            pl.BlockSpec((2, BLK, HH), lambda i: (0, i, 0)),
            pl.BlockSpec((2, BLK, HH), lambda i: (0, i, 0)),
            pl.BlockSpec((2, BLK, 16), lambda i: (0, i, 0)),
            pl.BlockSpec((1, H), lambda i: (0, 0)),
            pl.BlockSpec((BLK, 1), lambda i: (i, 0)),
            pl.BlockSpec((H, H), lambda i: (0, 0)),
            pl.BlockSpec((1, H), lambda i: (0, 0)),
            pl.BlockSpec((H, 1), lambda i: (0, 0)),
            pl.BlockSpec((1, 1), lambda i: (0, 0)),
        ],
        out_specs=pl.BlockSpec((G, 1), lambda i: (0, 0)),
        out_shape=jax.ShapeDtypeStruct((G, 1), jnp.float32),
        scratch_shapes=[pltpu.VMEM((G, H), jnp.float32)],
    )(parts, y2, degp, b, batch2d, w0, b0, w1, b1)


def kernel(x_p, edge_index_p, x_p_batch, W_conv0, b_conv0, W_conv1, b_conv1,
           W_conv2, b_conv2, W_lin0, b_lin0, W_lin1, b_lin1):
    src2d = edge_index_p[0].reshape(IDX_ROWS, CHUNK)
    dst2d = edge_index_p[1].reshape(IDX_ROWS, CHUNK)
    batch2d = x_p_batch.reshape(N, 1)
    ones_c = jnp.ones((CHUNK, 16), jnp.float32)
    zeros16 = jnp.zeros((TILE_SLICE, 16), jnp.float32)
    zeros64 = jnp.zeros((TILE_SLICE, HH), jnp.float32)
    b0 = b_conv0.reshape(1, H)
    b1 = b_conv1.reshape(1, H)
    b2 = b_conv2.reshape(1, H)
    bl0 = b_lin0.reshape(1, H)
    bl1 = b_lin1.reshape(1, 1)

    degp = _sc_degree(dst2d, ones_c, zeros16)
    xw0 = _tc_matmul(x_p, W_conv0)
    y0 = _tc_scale(xw0, degp)
    p0 = _sc_aggregate(y0, src2d, dst2d, zeros64)
    y1 = _tc_layer(p0, y0, degp, b0, W_conv1)
    p1 = _sc_aggregate(y1, src2d, dst2d, zeros64)
    y2 = _tc_layer(p1, y1, degp, b1, W_conv2)
    p2 = _sc_aggregate(y2, src2d, dst2d, zeros64)
    return _tc_final(p2, y2, degp, b2, batch2d, W_lin0, bl0, W_lin1, bl1)


# speedup vs baseline: 23.1151x; 1.1495x over previous
import functools

import jax
import jax.numpy as jnp
from jax import lax
from jax.experimental import pallas as pl
from jax.experimental.pallas import tpu as pltpu
from jax.experimental.pallas import tpu_sc as plsc

N = 10000
E = 320000
D = 128
H = 128
HH = H // 2
G = 64

N_PAD = 10240
CHUNK = 125
IDX_ROWS = E // CHUNK
DEG_ROWS = IDX_ROWS // 32
AGG_ROWS = IDX_ROWS // 16
TILE_SLICE = N_PAD // 16
NBUF = 4
BLK = 1000
NBLK = N // BLK

_mesh = plsc.VectorSubcoreMesh(core_axis_name="c", subcore_axis_name="s")
_sc_params = pltpu.CompilerParams(use_tc_tiling_on_sc=False)


def _sc_degree(dst2d, ones_c, zeros16):
    @functools.partial(
        pl.kernel,
        mesh=_mesh,
        out_type=jax.ShapeDtypeStruct((2, N_PAD, 16), jnp.float32),
        compiler_params=_sc_params,
        scratch_types=[
            pltpu.VMEM((DEG_ROWS, CHUNK), jnp.int32),
            pltpu.VMEM((CHUNK, 16), jnp.float32),
            pltpu.VMEM_SHARED((N_PAD, 16), jnp.float32),
        ],
    )
    def k(dst_hbm, ones_hbm, z_hbm, out_hbm, didx, ones_v, acc):
        cid = lax.axis_index("c")
        sid = lax.axis_index("s")
        wrow = (cid * 16 + sid) * DEG_ROWS
        pltpu.sync_copy(dst_hbm.at[pl.ds(wrow, DEG_ROWS)], didx)
        pltpu.sync_copy(ones_hbm, ones_v)
        pltpu.sync_copy(z_hbm, acc.at[pl.ds(sid * TILE_SLICE, TILE_SLICE)])
        plsc.subcore_barrier()

        @pl.loop(0, DEG_ROWS)
        def _(r):
            pltpu.sync_copy(ones_v, acc.at[didx.at[r]], add=True)

        plsc.subcore_barrier()
        pltpu.sync_copy(
            acc.at[pl.ds(sid * TILE_SLICE, TILE_SLICE)],
            out_hbm.at[cid, pl.ds(sid * TILE_SLICE, TILE_SLICE)],
        )

    return k(dst2d, ones_c, zeros16)


def _sc_aggregate(y2, src2d, dst2d, zeros64):
    @functools.partial(
        pl.kernel,
        mesh=_mesh,
        out_type=jax.ShapeDtypeStruct((2, N_PAD, HH), jnp.float32),
        compiler_params=_sc_params,
        scratch_types=(
            [pltpu.VMEM((AGG_ROWS, CHUNK), jnp.int32),
             pltpu.VMEM((AGG_ROWS, CHUNK), jnp.int32),
             pltpu.VMEM_SHARED((N_PAD, HH), jnp.float32)]
            + [pltpu.VMEM((CHUNK, HH), jnp.float32)] * NBUF
            + [pltpu.SemaphoreType.DMA] * (2 * NBUF)
        ),
    )
    def k(y_hbm, src_hbm, dst_hbm, z_hbm, out_hbm, sidx, didx, acc, *rest):
        bufs = rest[:NBUF]
        gsems = rest[NBUF:2 * NBUF]
        ssems = rest[2 * NBUF:]
        cid = lax.axis_index("c")
        sid = lax.axis_index("s")
        wrow = sid * AGG_ROWS
        yc = y_hbm.at[cid]
        pltpu.sync_copy(src_hbm.at[pl.ds(wrow, AGG_ROWS)], sidx)
        pltpu.sync_copy(dst_hbm.at[pl.ds(wrow, AGG_ROWS)], didx)
        pltpu.sync_copy(z_hbm, acc.at[pl.ds(sid * TILE_SLICE, TILE_SLICE)])
        plsc.subcore_barrier()

        for k in range(NBUF):
            pltpu.async_copy(yc.at[sidx.at[k]], bufs[k], gsems[k])

        @pl.loop(0, AGG_ROWS // NBUF)
        def _(j):
            r = j * NBUF
            scat = []
            for k in range(NBUF):
                pltpu.make_async_copy(
                    yc.at[sidx.at[r + k]], bufs[k], gsems[k]).wait()
                scat.append(pltpu.async_copy(
                    bufs[k], acc.at[didx.at[r + k]], ssems[k], add=True))
            for k in range(NBUF):
                scat[k].wait()

                @pl.when(r + k + NBUF < AGG_ROWS)
                def _():
                    pltpu.async_copy(
                        yc.at[sidx.at[r + k + NBUF]], bufs[k], gsems[k])

        plsc.subcore_barrier()
        pltpu.sync_copy(
            acc.at[pl.ds(sid * TILE_SLICE, TILE_SLICE)],
            out_hbm.at[cid, pl.ds(sid * TILE_SLICE, TILE_SLICE)],
        )

    return k(y2, src2d, dst2d, zeros64)


def _bdot(a, b):
    return lax.dot_general(a.astype(jnp.bfloat16), b.astype(jnp.bfloat16),
                           (((1,), (0,)), ((), ())),
                           preferred_element_type=jnp.float32)


def _dot_f32(a, b):
    return lax.dot_general(a, b, (((1,), (0,)), ((), ())),
                           precision=lax.Precision.HIGHEST,
                           preferred_element_type=jnp.float32)


def _split(o_ref, y):
    o_ref[0] = y[:, :HH]
    o_ref[1] = y[:, HH:]


def _tc_matmul(x, w):
    def body(x_ref, w_ref, o_ref):
        o_ref[...] = _bdot(x_ref[...], w_ref[...])

    return pl.pallas_call(
        body,
        grid=(NBLK,),
        in_specs=[
            pl.BlockSpec((BLK, D), lambda i: (i, 0)),
            pl.BlockSpec((D, H), lambda i: (0, 0)),
        ],
        out_specs=pl.BlockSpec((BLK, H), lambda i: (i, 0)),
        out_shape=jax.ShapeDtypeStruct((N, H), jnp.float32),
    )(x, w)


def _tc_scale(xw, degp):
    def body(xw_ref, d_ref, o_ref):
        deg = d_ref[0, :, :1] + d_ref[1, :, :1] + 1.0
        _split(o_ref, lax.rsqrt(deg) * xw_ref[...])

    return pl.pallas_call(
        body,
        grid=(NBLK,),
        in_specs=[
            pl.BlockSpec((BLK, H), lambda i: (i, 0)),
            pl.BlockSpec((2, BLK, 16), lambda i: (0, i, 0)),
        ],
        out_specs=pl.BlockSpec((2, BLK, HH), lambda i: (0, i, 0)),
        out_shape=jax.ShapeDtypeStruct((2, N, HH), jnp.float32),
    )(xw, degp)


def _tc_layer(parts, y2, degp, b, w_next):
    def body(p_ref, y_ref, d_ref, b_ref, w_ref, o_ref):
        deg = d_ref[0, :, :1] + d_ref[1, :, :1] + 1.0
        dinv = lax.rsqrt(deg)
        s = jnp.concatenate([p_ref[0] + y_ref[0], p_ref[1] + y_ref[1]], axis=1)
        h = jnp.maximum(dinv * s + b_ref[...], 0.0)
        _split(o_ref, dinv * _bdot(h, w_ref[...]))

    return pl.pallas_call(
        body,
        grid=(NBLK,),
        in_specs=[
            pl.BlockSpec((2, BLK, HH), lambda i: (0, i, 0)),
            pl.BlockSpec((2, BLK, HH), lambda i: (0, i, 0)),
            pl.BlockSpec((2, BLK, 16), lambda i: (0, i, 0)),
            pl.BlockSpec((1, H), lambda i: (0, 0)),
            pl.BlockSpec((H, H), lambda i: (0, 0)),
        ],
        out_specs=pl.BlockSpec((2, BLK, HH), lambda i: (0, i, 0)),
        out_shape=jax.ShapeDtypeStruct((2, N, HH), jnp.float32),
    )(parts, y2, degp, b, w_next)


def _tc_final(parts, y2, degp, b, batch2d, w0, b0, w1, b1):
    def body(p_ref, y_ref, d_ref, b_ref, bat_ref, w0_ref, b0_ref, w1_ref,
             b1_ref, o_ref, acc_ref):
        i = pl.program_id(0)

        @pl.when(i == 0)
        def _():
            acc_ref[...] = jnp.full((G, H), -jnp.inf, jnp.float32)

        deg = d_ref[0, :, :1] + d_ref[1, :, :1] + 1.0
        dinv = lax.rsqrt(deg)
        s = jnp.concatenate([p_ref[0] + y_ref[0], p_ref[1] + y_ref[1]], axis=1)
        h = jnp.maximum(dinv * s + b_ref[...], 0.0)
        bat = bat_ref[...]
        g_lo = bat_ref[0, 0]
        g_hi = bat_ref[BLK - 1, 0]
        seg_col = lax.broadcasted_iota(jnp.int32, (G, 1), 0)

        def seg_body(g, m):
            sel = jnp.where(bat == g, h, -jnp.inf)
            row = jnp.max(sel, axis=0, keepdims=True)
            return jnp.maximum(m, jnp.where(seg_col == g, row, -jnp.inf))

        acc_ref[...] = lax.fori_loop(g_lo, g_hi + 1, seg_body, acc_ref[...])

        @pl.when(i == NBLK - 1)
        def _():
            g1 = _dot_f32(acc_ref[...], w0_ref[...]) + b0_ref[...]
            o_ref[...] = _dot_f32(g1, w1_ref[...]) + b1_ref[...]

    return pl.pallas_call(
        body,
        grid=(NBLK,),
        in_specs=[
            pl.BlockSpec((2, BLK, HH), lambda i: (0, i, 0)),
            pl.BlockSpec((2, BLK, HH), lambda i: (0, i, 0)),
            pl.BlockSpec((2, BLK, 16), lambda i: (0, i, 0)),
            pl.BlockSpec((1, H), lambda i: (0, 0)),
            pl.BlockSpec((BLK, 1), lambda i: (i, 0)),
            pl.BlockSpec((H, H), lambda i: (0, 0)),
            pl.BlockSpec((1, H), lambda i: (0, 0)),
            pl.BlockSpec((H, 1), lambda i: (0, 0)),
            pl.BlockSpec((1, 1), lambda i: (0, 0)),
        ],
        out_specs=pl.BlockSpec((G, 1), lambda i: (0, 0)),
        out_shape=jax.ShapeDtypeStruct((G, 1), jnp.float32),
        scratch_shapes=[pltpu.VMEM((G, H), jnp.float32)],
    )(parts, y2, degp, b, batch2d, w0, b0, w1, b1)


def kernel(x_p, edge_index_p, x_p_batch, W_conv0, b_conv0, W_conv1, b_conv1,
           W_conv2, b_conv2, W_lin0, b_lin0, W_lin1, b_lin1):
    src2d = edge_index_p[0].reshape(IDX_ROWS, CHUNK)
    dst2d = edge_index_p[1].reshape(IDX_ROWS, CHUNK)
    batch2d = x_p_batch.reshape(N, 1)
    ones_c = jnp.ones((CHUNK, 16), jnp.float32)
    zeros16 = jnp.zeros((TILE_SLICE, 16), jnp.float32)
    zeros64 = jnp.zeros((TILE_SLICE, HH), jnp.float32)
    b0 = b_conv0.reshape(1, H)
    b1 = b_conv1.reshape(1, H)
    b2 = b_conv2.reshape(1, H)
    bl0 = b_lin0.reshape(1, H)
    bl1 = b_lin1.reshape(1, 1)

    degp = _sc_degree(dst2d, ones_c, zeros16)
    xw0 = _tc_matmul(x_p, W_conv0)
    y0 = _tc_scale(xw0, degp)
    p0 = _sc_aggregate(y0, src2d, dst2d, zeros64)
    y1 = _tc_layer(p0, y0, degp, b0, W_conv1)
    p1 = _sc_aggregate(y1, src2d, dst2d, zeros64)
    y2 = _tc_layer(p1, y1, degp, b1, W_conv2)
    p2 = _sc_aggregate(y2, src2d, dst2d, zeros64)
    return _tc_final(p2, y2, degp, b2, batch2d, W_lin0, bl0, W_lin1, bl1)
